# baseline (speedup 1.0000x reference)
import functools

import jax
import jax.numpy as jnp
from jax import lax
from jax.experimental import pallas as pl
from jax.experimental.pallas import tpu as pltpu
from jax.experimental.pallas import tpu_sc as plsc

VOCAB = 1000000
D = 32
B, T = 4096, 200
TOTAL = B * T
NW = 32
PER_W = TOTAL // NW
CHUNK = 1024
G = 128
SUBG = CHUNK // G
N_CHUNKS = PER_W // CHUNK

_LANES = 16


def _tanh16(x):
    t = jnp.exp(jnp.abs(x) * -2.0)
    r = (1.0 - t) / (1.0 + t)
    return jnp.where(x < 0.0, -r, r)


def _body(table_hbm, idx_hbm, out_hbm, idx_v, rows_v, sem):
    wid = lax.axis_index("s") * 2 + lax.axis_index("c")

    def chunk_step(c, _):
        base = wid * PER_W + c * CHUNK
        irow = wid * (PER_W // G) + c * SUBG

        pltpu.sync_copy(idx_hbm.at[pl.ds(irow, SUBG)], idx_v)
        descs = [
            pltpu.async_copy(
                table_hbm.at[idx_v.at[j]],
                rows_v.at[pl.ds(j * G, G)],
                sem,
            )
            for j in range(SUBG)
        ]
        for d in descs:
            d.wait()

        def row_step(i, _):
            for h in range(D // _LANES):
                sl = pl.ds(h * _LANES, _LANES)
                rows_v[i, sl] = _tanh16(rows_v[i, sl])
            return 0

        lax.fori_loop(0, CHUNK, row_step, 0)

        pltpu.sync_copy(rows_v, out_hbm.at[pl.ds(base, CHUNK)])
        return 0

    lax.fori_loop(0, N_CHUNKS, chunk_step, 0)


@jax.jit
def kernel(input_ids, table):
    idx = input_ids.astype(jnp.int32).reshape(TOTAL // G, G)
    mesh = plsc.VectorSubcoreMesh(core_axis_name="c", subcore_axis_name="s")
    out = pl.kernel(
        _body,
        out_type=jax.ShapeDtypeStruct((TOTAL, D), jnp.float32),
        mesh=mesh,
        compiler_params=pltpu.CompilerParams(use_tc_tiling_on_sc=False),
        scratch_types=[
            pltpu.VMEM((SUBG, G), jnp.int32),
            pltpu.VMEM((CHUNK, D), jnp.float32),
            pltpu.SemaphoreType.DMA,
        ],
    )(table, idx)
    return out.reshape(B, T, D)


# speedup vs baseline: 1.4244x; 1.4244x over previous
import functools

import jax
import jax.numpy as jnp
from jax import lax
from jax.experimental import pallas as pl
from jax.experimental.pallas import tpu as pltpu
from jax.experimental.pallas import tpu_sc as plsc

VOCAB = 1000000
D = 32
B, T = 4096, 200
TOTAL = B * T
NW = 32
PER_W = TOTAL // NW
CHUNK = 1024
G = 128
SUBG = CHUNK // G
N_CHUNKS = PER_W // CHUNK

_LANES = 16
_UNROLL = 8


def _tanh16(x):
    t = jnp.exp(x * -2.0)
    return 2.0 / (1.0 + t) - 1.0


def _body(table_hbm, idx_hbm, out_hbm, idx_v, rows_v, sem):
    wid = lax.axis_index("s") * 2 + lax.axis_index("c")

    def chunk_step(c, _):
        base = wid * PER_W + c * CHUNK
        irow = wid * (PER_W // G) + c * SUBG

        pltpu.sync_copy(idx_hbm.at[pl.ds(irow, SUBG)], idx_v)
        descs = [
            pltpu.async_copy(
                table_hbm.at[idx_v.at[j]],
                rows_v.at[pl.ds(j * G, G)],
                sem,
            )
            for j in range(SUBG)
        ]
        for d in descs:
            d.wait()

        def row_step(i, _):
            r0 = i * _UNROLL
            for u in range(_UNROLL):
                for h in range(D // _LANES):
                    sl = pl.ds(h * _LANES, _LANES)
                    rows_v[r0 + u, sl] = _tanh16(rows_v[r0 + u, sl])
            return 0

        lax.fori_loop(0, CHUNK // _UNROLL, row_step, 0)

        pltpu.sync_copy(rows_v, out_hbm.at[pl.ds(base, CHUNK)])
        return 0

    lax.fori_loop(0, N_CHUNKS, chunk_step, 0)


@jax.jit
def kernel(input_ids, table):
    idx = input_ids.astype(jnp.int32).reshape(TOTAL // G, G)
    mesh = plsc.VectorSubcoreMesh(core_axis_name="c", subcore_axis_name="s")
    out = pl.kernel(
        _body,
        out_type=jax.ShapeDtypeStruct((TOTAL, D), jnp.float32),
        mesh=mesh,
        compiler_params=pltpu.CompilerParams(use_tc_tiling_on_sc=False),
        scratch_types=[
            pltpu.VMEM((SUBG, G), jnp.int32),
            pltpu.VMEM((CHUNK, D), jnp.float32),
            pltpu.SemaphoreType.DMA,
        ],
    )(table, idx)
    return out.reshape(B, T, D)
